# Initial kernel scaffold; baseline (speedup 1.0000x reference)
import jax
import jax.numpy as jnp
from jax.experimental import pallas as pl


def kernel(x, W1, g1, b1, Wf, gf, bf, Wc, gc, bc):
    raise NotImplementedError("write your pallas kernel here")


# speedup vs baseline: 12.6755x; 12.6755x over previous
import functools

import jax
import jax.numpy as jnp
from jax import lax
from jax.experimental import pallas as pl
from jax.experimental.pallas import tpu as pltpu
from jax.experimental.pallas import tpu_sc as plsc

B = 8
N = 2048
KNN = 20
OUT_K = 16
OUT_DIM = 128
NPOINT = N // 2
EPS = 1e-5
SLOPE = 0.2
BN_ROWS = B * N
BM_ROWS = B * NPOINT
NEG = -1e30

_PREC = jax.lax.Precision.HIGHEST


_QB = 256


def _k1_body(xp_ref, xptb_ref, xpb_ref, wa_ref, wv_ref,
             uu_ref, v_ref, idx_ref):
    b = pl.program_id(0)
    xp = xp_ref[0]
    xptb = xptb_ref[0]
    xpb = xpb_ref[0]
    u = jnp.dot(xpb, wa_ref[...], preferred_element_type=jnp.float32,
                precision=_PREC)
    uu_ref[0, :, :64] = u
    uu_ref[0, :, 64:] = u * u
    v_ref[0] = jnp.dot(xpb, wv_ref[...], preferred_element_type=jnp.float32,
                       precision=_PREC)
    g = jnp.dot(xp.astype(jnp.bfloat16), xptb.astype(jnp.bfloat16),
                preferred_element_type=jnp.float32)
    xxc = jnp.sum(xp * xp, axis=1, keepdims=True)
    xxr = jnp.sum(xptb * xptb, axis=0, keepdims=True)
    m = 2.0 * g - xxr - xxc
    jrow = lax.broadcasted_iota(jnp.int32, (N, _QB), 0)
    base = b * N
    for k in range(KNN):
        mx = jnp.max(m, axis=0, keepdims=True)
        oh = m == mx
        idxk = jnp.max(jnp.where(oh, jrow, -1), axis=0)
        idx_ref[0, k] = idxk + base
        m = jnp.where(oh, NEG, m)


def _k1(xp, xpt, wa, wv):
    return pl.pallas_call(
        _k1_body,
        grid=(B, N // _QB),
        in_specs=[
            pl.BlockSpec((1, N, 8), lambda b, i: (b, 0, 0)),
            pl.BlockSpec((1, 8, _QB), lambda b, i: (b, 0, i)),
            pl.BlockSpec((1, _QB, 8), lambda b, i: (b, i, 0)),
            pl.BlockSpec((8, 64), lambda b, i: (0, 0)),
            pl.BlockSpec((8, 64), lambda b, i: (0, 0)),
        ],
        out_specs=[
            pl.BlockSpec((1, _QB, 128), lambda b, i: (b, i, 0)),
            pl.BlockSpec((1, _QB, 64), lambda b, i: (b, i, 0)),
            pl.BlockSpec((1, KNN, _QB), lambda b, i: (b, 0, i)),
        ],
        out_shape=[
            jax.ShapeDtypeStruct((B, N, 128), jnp.float32),
            jax.ShapeDtypeStruct((B, N, 64), jnp.float32),
            jax.ShapeDtypeStruct((B, KNN, N), jnp.int32),
        ],
    )(xp, xpt, xp, wa, wv)


_NW = 32
_C2 = 4
_PPW2 = BN_ROWS // _NW


def _k2_body(u_hbm, idx_hbm, smax_hbm, s1_hbm, s2_hbm,
             idx_v, rows_v, mx_v, s1_v, s2_v, sem):
    wid = lax.axis_index("s") * 2 + lax.axis_index("c")

    def chunk(ci, _):
        base = wid * _PPW2 + ci * _C2
        pltpu.sync_copy(idx_hbm.at[pl.ds(base * KNN, _C2 * KNN)], idx_v)
        pltpu.async_copy(u_hbm.at[idx_v], rows_v, sem).wait()

        def point(p, _):
            for cg in range(4):
                sl = pl.ds(cg * 16, 16)
                sq = pl.ds(64 + cg * 16, 16)
                amx = rows_v[p * KNN, sl]
                asum = amx
                asq = rows_v[p * KNN, sq]
                for k in range(1, KNN):
                    r = rows_v[p * KNN + k, sl]
                    amx = jnp.maximum(amx, r)
                    asum = asum + r
                    asq = asq + rows_v[p * KNN + k, sq]
                mx_v[p, sl] = amx
                s1_v[p, sl] = asum
                s2_v[p, sl] = asq
            return 0

        lax.fori_loop(0, _C2, point, 0)
        pltpu.sync_copy(mx_v, smax_hbm.at[pl.ds(base, _C2)])
        pltpu.sync_copy(s1_v, s1_hbm.at[pl.ds(base, _C2)])
        pltpu.sync_copy(s2_v, s2_hbm.at[pl.ds(base, _C2)])
        return 0

    lax.fori_loop(0, _PPW2 // _C2, chunk, 0)


_k2 = functools.partial(
    pl.kernel,
    mesh=plsc.VectorSubcoreMesh(core_axis_name="c", subcore_axis_name="s"),
    out_type=[
        jax.ShapeDtypeStruct((BN_ROWS, 64), jnp.float32),
        jax.ShapeDtypeStruct((BN_ROWS, 64), jnp.float32),
        jax.ShapeDtypeStruct((BN_ROWS, 64), jnp.float32),
    ],
    scratch_types=[
        pltpu.VMEM((_C2 * KNN,), jnp.int32),
        pltpu.VMEM((_C2 * KNN, 128), jnp.float32),
        pltpu.VMEM((_C2, 64), jnp.float32),
        pltpu.VMEM((_C2, 64), jnp.float32),
        pltpu.VMEM((_C2, 64), jnp.float32),
        pltpu.SemaphoreType.DMA,
    ],
)(_k2_body)


def _k3_body(smax_ref, s1_ref, s2_ref, v_ref, g1_ref, b1_ref, x1_ref):
    s1 = s1_ref[...]
    v = v_ref[...]
    cnt = float(BN_ROWS * KNN)
    sum_s1 = jnp.sum(s1, axis=0, keepdims=True)
    sum_v = jnp.sum(v, axis=0, keepdims=True)
    sum_s2 = jnp.sum(s2_ref[...], axis=0, keepdims=True)
    sum_vs1 = jnp.sum(v * s1, axis=0, keepdims=True)
    sum_v2 = jnp.sum(v * v, axis=0, keepdims=True)
    mean = (sum_s1 + KNN * sum_v) * (1.0 / cnt)
    e2 = (sum_s2 + 2.0 * sum_vs1 + KNN * sum_v2) * (1.0 / cnt)
    var = e2 - mean * mean
    rstd = lax.rsqrt(var + EPS)
    pre = (smax_ref[...] + v - mean) * (rstd * g1_ref[...]) + b1_ref[...]
    x1_ref[:, :64] = jnp.where(pre >= 0, pre, SLOPE * pre)
    x1_ref[:, 64:] = jnp.zeros((BN_ROWS, 64), jnp.float32)


def _k3(smax, s1, s2, v_rows, g1r, b1r):
    return pl.pallas_call(
        _k3_body,
        out_shape=jax.ShapeDtypeStruct((BN_ROWS, 128), jnp.float32),
    )(smax, s1, s2, v_rows, g1r, b1r)


def _k4_body(xf_ref, xp_ref, p1_ref, sel_ref):
    x0 = xf_ref[0]
    x1 = xf_ref[1]
    x2 = xf_ref[2]
    lane = lax.broadcasted_iota(jnp.int32, (B, N), 1)
    zero = x0 * 0.0
    cent0 = (lane == 0).astype(jnp.float32) + zero
    dist0 = zero + 1e10
    sel0 = zero.astype(jnp.int32) - 1

    def step(t, carry):
        cent_oh, dist, sel = carry
        sel = jnp.where(cent_oh > 0.5, t, sel)
        c0 = jnp.sum(cent_oh * x0, axis=1, keepdims=True)
        c1 = jnp.sum(cent_oh * x1, axis=1, keepdims=True)
        c2 = jnp.sum(cent_oh * x2, axis=1, keepdims=True)
        d0 = x0 - c0
        d1 = x1 - c1
        d2 = x2 - c2
        d = d0 * d0 + d1 * d1 + d2 * d2
        dist = jnp.minimum(dist, d)
        gm = jnp.max(dist, axis=1, keepdims=True)
        eq = dist == gm
        fidx = jnp.min(jnp.where(eq, lane, N), axis=1, keepdims=True)
        return ((lane == fidx).astype(jnp.float32), dist, sel)

    _, _, sel = lax.fori_loop(0, NPOINT, step, (cent0, dist0, sel0))
    sel_ref[...] = sel

    trow = lax.broadcasted_iota(jnp.int32, (NPOINT, N), 0)
    for b in range(B):
        perm = (trow == sel_ref[b:b + 1, :]).astype(jnp.float32)
        p1_ref[b] = jnp.dot(perm, xp_ref[b],
                            preferred_element_type=jnp.float32,
                            precision=_PREC)


def _k4(xf, xp):
    return pl.pallas_call(
        _k4_body,
        out_shape=jax.ShapeDtypeStruct((B, NPOINT, 8), jnp.float32),
        scratch_shapes=[pltpu.VMEM((B, N), jnp.int32)],
    )(xf, xp)


def _k5_body(xp_ref, pptb_ref, idx_ref):
    b = pl.program_id(0)
    xp = xp_ref[0]
    pptb = pptb_ref[0]
    g = jnp.dot(xp.astype(jnp.bfloat16), pptb.astype(jnp.bfloat16),
                preferred_element_type=jnp.float32)
    xxc = jnp.sum(xp * xp, axis=1, keepdims=True)
    ppr = jnp.sum(pptb * pptb, axis=0, keepdims=True)
    d2 = xxc + ppr - 2.0 * g
    m = -jnp.sqrt(jnp.maximum(d2, 0.0))
    jrow = lax.broadcasted_iota(jnp.int32, (N, _QB), 0)
    base = b * N
    for k in range(OUT_K):
        mx = jnp.max(m, axis=0, keepdims=True)
        oh = m == mx
        idxk = jnp.max(jnp.where(oh, jrow, -1), axis=0)
        idx_ref[0, k] = idxk + base
        m = jnp.where(oh, NEG, m)


def _k5(xp, ppt):
    return pl.pallas_call(
        _k5_body,
        grid=(B, NPOINT // _QB),
        in_specs=[
            pl.BlockSpec((1, N, 8), lambda b, i: (b, 0, 0)),
            pl.BlockSpec((1, 8, _QB), lambda b, i: (b, 0, i)),
        ],
        out_specs=pl.BlockSpec((1, OUT_K, _QB), lambda b, i: (b, 0, i)),
        out_shape=jax.ShapeDtypeStruct((B, OUT_K, NPOINT), jnp.int32),
    )(xp, ppt)


_C6 = 8
_PPW6 = BM_ROWS // _NW


def _k6_body(x1_hbm, idx_hbm, out_hbm, idx_v, rows_v, acc_v, sem):
    wid = lax.axis_index("s") * 2 + lax.axis_index("c")

    def chunk(ci, _):
        base = wid * _PPW6 + ci * _C6
        pltpu.sync_copy(idx_hbm.at[pl.ds(base * OUT_K, _C6 * OUT_K)], idx_v)
        pltpu.async_copy(x1_hbm.at[idx_v], rows_v, sem).wait()

        def point(p, _):
            for cg in range(4):
                sl = pl.ds(cg * 16, 16)
                asum = rows_v[p * OUT_K, sl]
                for k in range(1, OUT_K):
                    asum = asum + rows_v[p * OUT_K + k, sl]
                acc_v[p, sl] = asum * (1.0 / OUT_K)
            return 0

        lax.fori_loop(0, _C6, point, 0)
        pltpu.sync_copy(acc_v, out_hbm.at[pl.ds(base, _C6)])
        return 0

    lax.fori_loop(0, _PPW6 // _C6, chunk, 0)


_k6 = functools.partial(
    pl.kernel,
    mesh=plsc.VectorSubcoreMesh(core_axis_name="c", subcore_axis_name="s"),
    out_type=jax.ShapeDtypeStruct((BM_ROWS, 64), jnp.float32),
    scratch_types=[
        pltpu.VMEM((_C6 * OUT_K,), jnp.int32),
        pltpu.VMEM((_C6 * OUT_K, 128), jnp.float32),
        pltpu.VMEM((_C6, 64), jnp.float32),
        pltpu.SemaphoreType.DMA,
    ],
)(_k6_body)


def _k7_body(ebdst_ref, wf_ref, wc_ref, gf_ref, bf_ref, gc_ref, bc_ref,
             f_ref, c_ref):
    ebdst = ebdst_ref[...]

    def head(w_ref, g_ref, b_ref, o_ref):
        z = jnp.dot(w_ref[...], ebdst, preferred_element_type=jnp.float32,
                    precision=_PREC)
        mu = jnp.mean(z, axis=1, keepdims=True)
        zc = z - mu
        va = jnp.mean(zc * zc, axis=1, keepdims=True)
        o_ref[...] = zc * (lax.rsqrt(va + EPS) * g_ref[...]) + b_ref[...]

    head(wf_ref, gf_ref, bf_ref, f_ref)
    head(wc_ref, gc_ref, bc_ref, c_ref)


def _k7(ebdst, wf, wc, gfr, bfr, gcr, bcr):
    return pl.pallas_call(
        _k7_body,
        out_shape=[
            jax.ShapeDtypeStruct((OUT_DIM, BM_ROWS), jnp.float32),
            jax.ShapeDtypeStruct((OUT_DIM, BM_ROWS), jnp.float32),
        ],
    )(ebdst, wf, wc, gfr, bfr, gcr, bcr)


def kernel(x, W1, g1, b1, Wf, gf, bf, Wc, gc, bc):
    x = x.astype(jnp.float32)
    xp = jnp.pad(x, ((0, 0), (0, 0), (0, 5)))
    xpt = jnp.transpose(xp, (0, 2, 1))
    wa = jnp.pad(jnp.transpose(W1[:, :3]), ((0, 5), (0, 0)))
    wv = jnp.pad(jnp.transpose(W1[:, 3:] - W1[:, :3]), ((0, 5), (0, 0)))

    uu, v, idx = _k1(xp, xpt, wa, wv)
    uu_rows = uu.reshape(BN_ROWS, 128)
    v_rows = v.reshape(BN_ROWS, 64)
    idx_flat = jnp.transpose(idx, (0, 2, 1)).reshape(BN_ROWS * KNN)

    smax, s1, s2 = _k2(uu_rows, idx_flat)
    x1_rows = _k3(smax, s1, s2, v_rows,
                  g1.reshape(1, 64), b1.reshape(1, 64))

    xf = jnp.transpose(xp[:, :, :3], (2, 0, 1))
    p1p = _k4(xf, xp)
    p1 = p1p[:, :, :3]

    ppt = jnp.transpose(p1p, (0, 2, 1))
    idx2 = _k5(xp, ppt)
    idx2_flat = jnp.transpose(idx2, (0, 2, 1)).reshape(BM_ROWS * OUT_K)

    ebds = _k6(x1_rows, idx2_flat)
    ebdst = jnp.transpose(ebds)

    fo, co = _k7(ebdst, Wf, Wc,
                 gf.reshape(OUT_DIM, 1), bf.reshape(OUT_DIM, 1),
                 gc.reshape(OUT_DIM, 1), bc.reshape(OUT_DIM, 1))
    f_out = jnp.transpose(fo.reshape(OUT_DIM, B, NPOINT), (1, 0, 2))
    c_out = jnp.transpose(co.reshape(OUT_DIM, B, NPOINT), (1, 0, 2))

    mask = jnp.ones((B, NPOINT), dtype=bool)
    return (f_out, c_out, p1, p1, mask, mask)


# speedup vs baseline: 12.8555x; 1.0142x over previous
import functools

import jax
import jax.numpy as jnp
from jax import lax
from jax.experimental import pallas as pl
from jax.experimental.pallas import tpu as pltpu
from jax.experimental.pallas import tpu_sc as plsc

B = 8
N = 2048
KNN = 20
OUT_K = 16
OUT_DIM = 128
NPOINT = N // 2
EPS = 1e-5
SLOPE = 0.2
BN_ROWS = B * N
BM_ROWS = B * NPOINT
NEG = -1e30

_PREC = jax.lax.Precision.HIGHEST


_QB = 256


def _k1_body(xp_ref, xptb_ref, xpb_ref, wa_ref, wv_ref,
             uu_ref, v_ref, idx_ref):
    b = pl.program_id(0)
    xp = xp_ref[0]
    xptb = xptb_ref[0]
    xpb = xpb_ref[0]
    u = jnp.dot(xpb, wa_ref[...], preferred_element_type=jnp.float32,
                precision=_PREC)
    uu_ref[0, :, :64] = u
    uu_ref[0, :, 64:] = u * u
    v_ref[0] = jnp.dot(xpb, wv_ref[...], preferred_element_type=jnp.float32,
                       precision=_PREC)
    g = jnp.dot(xp.astype(jnp.bfloat16), xptb.astype(jnp.bfloat16),
                preferred_element_type=jnp.float32)
    xxc = jnp.sum(xp * xp, axis=1, keepdims=True)
    xxr = jnp.sum(xptb * xptb, axis=0, keepdims=True)
    m = 2.0 * g - xxr - xxc
    jrow = lax.broadcasted_iota(jnp.int32, (N, _QB), 0)
    base = b * N
    for k in range(KNN):
        mx = jnp.max(m, axis=0, keepdims=True)
        oh = m == mx
        idxk = jnp.max(jnp.where(oh, jrow, -1), axis=0)
        idx_ref[0, k] = idxk + base
        m = jnp.where(oh, NEG, m)


def _k1(xp, xpt, wa, wv):
    return pl.pallas_call(
        _k1_body,
        grid=(B, N // _QB),
        in_specs=[
            pl.BlockSpec((1, N, 8), lambda b, i: (b, 0, 0)),
            pl.BlockSpec((1, 8, _QB), lambda b, i: (b, 0, i)),
            pl.BlockSpec((1, _QB, 8), lambda b, i: (b, i, 0)),
            pl.BlockSpec((8, 64), lambda b, i: (0, 0)),
            pl.BlockSpec((8, 64), lambda b, i: (0, 0)),
        ],
        out_specs=[
            pl.BlockSpec((1, _QB, 128), lambda b, i: (b, i, 0)),
            pl.BlockSpec((1, _QB, 64), lambda b, i: (b, i, 0)),
            pl.BlockSpec((1, KNN, _QB), lambda b, i: (b, 0, i)),
        ],
        out_shape=[
            jax.ShapeDtypeStruct((B, N, 128), jnp.float32),
            jax.ShapeDtypeStruct((B, N, 64), jnp.float32),
            jax.ShapeDtypeStruct((B, KNN, N), jnp.int32),
        ],
    )(xp, xpt, xp, wa, wv)


_NW = 32
_C2 = 4
_PPW2 = BN_ROWS // _NW


_NCH2 = _PPW2 // _C2
_GRP = 4
_SEG2 = 4


def _k2_body(u_hbm, idx2d_hbm, ms1_hbm, s2_hbm,
             idx_v, rows0, rows1, rows2, rows3, ms1_v, s2_v, sem):
    wid = lax.axis_index("s") * 2 + lax.axis_index("c")
    rows = [rows0, rows1, rows2, rows3]
    pltpu.sync_copy(idx2d_hbm.at[pl.ds(wid * _NCH2, _NCH2)], idx_v)

    for seg in range(_SEG2):
        def group(g, _):
            cbase = seg * (_NCH2 // _SEG2) + g * _GRP
            descs = [
                pltpu.async_copy(u_hbm.at[idx_v.at[cbase + j]], rows[j], sem)
                for j in range(_GRP)
            ]
            for d in descs:
                d.wait()
            for j in range(_GRP):

                def point(p, _, rv=rows[j], cj=cbase + j):
                    o = (cj - seg * (_NCH2 // _SEG2)) * _C2 + p
                    for cg in range(4):
                        sl = pl.ds(cg * 16, 16)
                        sq = pl.ds(64 + cg * 16, 16)
                        amx = rv[p * KNN, sl]
                        asum = amx
                        asq = rv[p * KNN, sq]
                        for k in range(1, KNN):
                            r = rv[p * KNN + k, sl]
                            amx = jnp.maximum(amx, r)
                            asum = asum + r
                            asq = asq + rv[p * KNN + k, sq]
                        ms1_v[o, sl] = amx
                        ms1_v[o, sq] = asum
                        s2_v[o, sl] = asq
                    return 0

                lax.fori_loop(0, _C2, point, 0)
            return 0

        lax.fori_loop(0, _NCH2 // _SEG2 // _GRP, group, 0)
        segrows = _PPW2 // _SEG2
        base = wid * _PPW2 + seg * segrows
        pltpu.sync_copy(ms1_v, ms1_hbm.at[pl.ds(base, segrows)])
        pltpu.sync_copy(s2_v, s2_hbm.at[pl.ds(base, segrows)])


_k2 = functools.partial(
    pl.kernel,
    mesh=plsc.VectorSubcoreMesh(core_axis_name="c", subcore_axis_name="s"),
    out_type=[
        jax.ShapeDtypeStruct((BN_ROWS, 128), jnp.float32),
        jax.ShapeDtypeStruct((BN_ROWS, 64), jnp.float32),
    ],
    scratch_types=[
        pltpu.VMEM((_NCH2, _C2 * KNN), jnp.int32),
        pltpu.VMEM((_C2 * KNN, 128), jnp.float32),
        pltpu.VMEM((_C2 * KNN, 128), jnp.float32),
        pltpu.VMEM((_C2 * KNN, 128), jnp.float32),
        pltpu.VMEM((_C2 * KNN, 128), jnp.float32),
        pltpu.VMEM((_PPW2 // _SEG2, 128), jnp.float32),
        pltpu.VMEM((_PPW2 // _SEG2, 64), jnp.float32),
        pltpu.SemaphoreType.DMA,
    ],
)(_k2_body)


def _k3_body(smax_ref, s1_ref, s2_ref, v_ref, g1_ref, b1_ref, x1_ref):
    s1 = s1_ref[...]
    v = v_ref[...]
    cnt = float(BN_ROWS * KNN)
    sum_s1 = jnp.sum(s1, axis=0, keepdims=True)
    sum_v = jnp.sum(v, axis=0, keepdims=True)
    sum_s2 = jnp.sum(s2_ref[...], axis=0, keepdims=True)
    sum_vs1 = jnp.sum(v * s1, axis=0, keepdims=True)
    sum_v2 = jnp.sum(v * v, axis=0, keepdims=True)
    mean = (sum_s1 + KNN * sum_v) * (1.0 / cnt)
    e2 = (sum_s2 + 2.0 * sum_vs1 + KNN * sum_v2) * (1.0 / cnt)
    var = e2 - mean * mean
    rstd = lax.rsqrt(var + EPS)
    pre = (smax_ref[...] + v - mean) * (rstd * g1_ref[...]) + b1_ref[...]
    x1_ref[:, :64] = jnp.where(pre >= 0, pre, SLOPE * pre)
    x1_ref[:, 64:] = jnp.zeros((BN_ROWS, 64), jnp.float32)


def _k3(smax, s1, s2, v_rows, g1r, b1r):
    return pl.pallas_call(
        _k3_body,
        out_shape=jax.ShapeDtypeStruct((BN_ROWS, 128), jnp.float32),
    )(smax, s1, s2, v_rows, g1r, b1r)


def _k4_body(xf_ref, xp_ref, p1_ref, sel_ref):
    x0 = xf_ref[0]
    x1 = xf_ref[1]
    x2 = xf_ref[2]
    lane = lax.broadcasted_iota(jnp.int32, (B, N), 1)
    zero = x0 * 0.0
    cent0 = (lane == 0).astype(jnp.float32) + zero
    dist0 = zero + 1e10
    sel0 = zero.astype(jnp.int32) - 1

    def step(t, carry):
        cent_oh, dist, sel = carry
        sel = jnp.where(cent_oh > 0.5, t, sel)
        c0 = jnp.sum(cent_oh * x0, axis=1, keepdims=True)
        c1 = jnp.sum(cent_oh * x1, axis=1, keepdims=True)
        c2 = jnp.sum(cent_oh * x2, axis=1, keepdims=True)
        d0 = x0 - c0
        d1 = x1 - c1
        d2 = x2 - c2
        d = d0 * d0 + d1 * d1 + d2 * d2
        dist = jnp.minimum(dist, d)
        gm = jnp.max(dist, axis=1, keepdims=True)
        eq = dist == gm
        fidx = jnp.min(jnp.where(eq, lane, N), axis=1, keepdims=True)
        return ((lane == fidx).astype(jnp.float32), dist, sel)

    _, _, sel = lax.fori_loop(0, NPOINT, step, (cent0, dist0, sel0))
    sel_ref[...] = sel

    trow = lax.broadcasted_iota(jnp.int32, (NPOINT, N), 0)
    for b in range(B):
        perm = (trow == sel_ref[b:b + 1, :]).astype(jnp.float32)
        p1_ref[b] = jnp.dot(perm, xp_ref[b],
                            preferred_element_type=jnp.float32,
                            precision=_PREC)


def _k4(xf, xp):
    return pl.pallas_call(
        _k4_body,
        out_shape=jax.ShapeDtypeStruct((B, NPOINT, 8), jnp.float32),
        scratch_shapes=[pltpu.VMEM((B, N), jnp.int32)],
    )(xf, xp)


def _k5_body(xp_ref, pptb_ref, idx_ref):
    b = pl.program_id(0)
    xp = xp_ref[0]
    pptb = pptb_ref[0]
    g = jnp.dot(xp.astype(jnp.bfloat16), pptb.astype(jnp.bfloat16),
                preferred_element_type=jnp.float32)
    xxc = jnp.sum(xp * xp, axis=1, keepdims=True)
    ppr = jnp.sum(pptb * pptb, axis=0, keepdims=True)
    d2 = xxc + ppr - 2.0 * g
    m = -jnp.sqrt(jnp.maximum(d2, 0.0))
    jrow = lax.broadcasted_iota(jnp.int32, (N, _QB), 0)
    base = b * N
    for k in range(OUT_K):
        mx = jnp.max(m, axis=0, keepdims=True)
        oh = m == mx
        idxk = jnp.max(jnp.where(oh, jrow, -1), axis=0)
        idx_ref[0, k] = idxk + base
        m = jnp.where(oh, NEG, m)


def _k5(xp, ppt):
    return pl.pallas_call(
        _k5_body,
        grid=(B, NPOINT // _QB),
        in_specs=[
            pl.BlockSpec((1, N, 8), lambda b, i: (b, 0, 0)),
            pl.BlockSpec((1, 8, _QB), lambda b, i: (b, 0, i)),
        ],
        out_specs=pl.BlockSpec((1, OUT_K, _QB), lambda b, i: (b, 0, i)),
        out_shape=jax.ShapeDtypeStruct((B, OUT_K, NPOINT), jnp.int32),
    )(xp, ppt)


_C6 = 8
_PPW6 = BM_ROWS // _NW


_NCH6 = _PPW6 // _C6


def _k6_body(x1_hbm, idx2d_hbm, out_hbm,
             idx_v, rows0, rows1, rows2, rows3, acc_v, sem):
    wid = lax.axis_index("s") * 2 + lax.axis_index("c")
    rows = [rows0, rows1, rows2, rows3]
    pltpu.sync_copy(idx2d_hbm.at[pl.ds(wid * _NCH6, _NCH6)], idx_v)

    def group(g, _):
        cbase = g * _GRP
        descs = [
            pltpu.async_copy(x1_hbm.at[idx_v.at[cbase + j]], rows[j], sem)
            for j in range(_GRP)
        ]
        for d in descs:
            d.wait()
        for j in range(_GRP):

            def point(p, _, rv=rows[j], cj=cbase + j):
                o = cj * _C6 + p
                for cg in range(4):
                    sl = pl.ds(cg * 16, 16)
                    asum = rv[p * OUT_K, sl]
                    for k in range(1, OUT_K):
                        asum = asum + rv[p * OUT_K + k, sl]
                    acc_v[o, sl] = asum * (1.0 / OUT_K)
                return 0

            lax.fori_loop(0, _C6, point, 0)
        return 0

    lax.fori_loop(0, _NCH6 // _GRP, group, 0)
    pltpu.sync_copy(acc_v, out_hbm.at[pl.ds(wid * _PPW6, _PPW6)])


_k6 = functools.partial(
    pl.kernel,
    mesh=plsc.VectorSubcoreMesh(core_axis_name="c", subcore_axis_name="s"),
    out_type=jax.ShapeDtypeStruct((BM_ROWS, 64), jnp.float32),
    scratch_types=[
        pltpu.VMEM((_NCH6, _C6 * OUT_K), jnp.int32),
        pltpu.VMEM((_C6 * OUT_K, 128), jnp.float32),
        pltpu.VMEM((_C6 * OUT_K, 128), jnp.float32),
        pltpu.VMEM((_C6 * OUT_K, 128), jnp.float32),
        pltpu.VMEM((_C6 * OUT_K, 128), jnp.float32),
        pltpu.VMEM((_PPW6, 64), jnp.float32),
        pltpu.SemaphoreType.DMA,
    ],
)(_k6_body)


def _k7_body(ebdst_ref, wf_ref, wc_ref, gf_ref, bf_ref, gc_ref, bc_ref,
             f_ref, c_ref):
    ebdst = ebdst_ref[...]

    def head(w_ref, g_ref, b_ref, o_ref):
        z = jnp.dot(w_ref[...], ebdst, preferred_element_type=jnp.float32,
                    precision=_PREC)
        mu = jnp.mean(z, axis=1, keepdims=True)
        zc = z - mu
        va = jnp.mean(zc * zc, axis=1, keepdims=True)
        o_ref[...] = zc * (lax.rsqrt(va + EPS) * g_ref[...]) + b_ref[...]

    head(wf_ref, gf_ref, bf_ref, f_ref)
    head(wc_ref, gc_ref, bc_ref, c_ref)


def _k7(ebdst, wf, wc, gfr, bfr, gcr, bcr):
    return pl.pallas_call(
        _k7_body,
        out_shape=[
            jax.ShapeDtypeStruct((OUT_DIM, BM_ROWS), jnp.float32),
            jax.ShapeDtypeStruct((OUT_DIM, BM_ROWS), jnp.float32),
        ],
    )(ebdst, wf, wc, gfr, bfr, gcr, bcr)


def kernel(x, W1, g1, b1, Wf, gf, bf, Wc, gc, bc):
    x = x.astype(jnp.float32)
    xp = jnp.pad(x, ((0, 0), (0, 0), (0, 5)))
    xpt = jnp.transpose(xp, (0, 2, 1))
    wa = jnp.pad(jnp.transpose(W1[:, :3]), ((0, 5), (0, 0)))
    wv = jnp.pad(jnp.transpose(W1[:, 3:] - W1[:, :3]), ((0, 5), (0, 0)))

    uu, v, idx = _k1(xp, xpt, wa, wv)
    uu_rows = uu.reshape(BN_ROWS, 128)
    v_rows = v.reshape(BN_ROWS, 64)
    idx2d = jnp.transpose(idx, (0, 2, 1)).reshape(
        BN_ROWS // _C2, _C2 * KNN)

    ms1, s2 = _k2(uu_rows, idx2d)
    smax, s1 = ms1[:, :64], ms1[:, 64:]
    x1_rows = _k3(smax, s1, s2, v_rows,
                  g1.reshape(1, 64), b1.reshape(1, 64))

    xf = jnp.transpose(xp[:, :, :3], (2, 0, 1))
    p1p = _k4(xf, xp)
    p1 = p1p[:, :, :3]

    ppt = jnp.transpose(p1p, (0, 2, 1))
    idx2 = _k5(xp, ppt)
    idx2d6 = jnp.transpose(idx2, (0, 2, 1)).reshape(
        BM_ROWS // _C6, _C6 * OUT_K)

    ebds = _k6(x1_rows, idx2d6)
    ebdst = jnp.transpose(ebds)

    fo, co = _k7(ebdst, Wf, Wc,
                 gf.reshape(OUT_DIM, 1), bf.reshape(OUT_DIM, 1),
                 gc.reshape(OUT_DIM, 1), bc.reshape(OUT_DIM, 1))
    f_out = jnp.transpose(fo.reshape(OUT_DIM, B, NPOINT), (1, 0, 2))
    c_out = jnp.transpose(co.reshape(OUT_DIM, B, NPOINT), (1, 0, 2))

    mask = jnp.ones((B, NPOINT), dtype=bool)
    return (f_out, c_out, p1, p1, mask, mask)
